# baseline (speedup 1.0000x reference)
import functools

import jax
import jax.numpy as jnp
from jax import lax
from jax.experimental import pallas as pl
from jax.experimental.pallas import tpu as pltpu
from jax.experimental.pallas import tpu_sc as plsc

B = 128
V = 100000

VSC = 25600
VT = V - VSC

NBLK = 20
CV = VT // NBLK
assert CV * NBLK == VT and CV % 8 == 0

NW = 32
RW = VSC // NW
NCH = 2
CHR = RW // NCH
assert RW * NW == VSC and CHR * NCH == RW
G = B // 16


def _tc_body(tok_ref, hbm_ref, out_ref, win_ref, wsem, *scratch):
    bufs = scratch[:NBLK]
    sems = scratch[NBLK:]
    wdescs = []
    for b in range(B):
        t = tok_ref[b]
        d = pltpu.make_async_copy(
            hbm_ref.at[pl.ds(t, 1), :], win_ref.at[pl.ds(b, 1), :], wsem
        )
        d.start()
        wdescs.append(d)
    descs = [
        pltpu.make_async_copy(hbm_ref.at[pl.ds(j * CV, CV), :], bufs[j], sems[j])
        for j in range(NBLK)
    ]
    for d in descs:
        d.start()
    for d in wdescs:
        d.wait()
    lane = lax.broadcasted_iota(jnp.int32, (1, B), 1)
    x = jnp.zeros((1, B), jnp.float32)
    for b in range(B):
        x = x + jnp.where(lane == b, win_ref[pl.ds(b, 1), :], 0.0)
    acc = jnp.zeros((1, B), jnp.int32)
    for j in range(NBLK):
        descs[j].wait()
        blk = bufs[j][...]
        acc = acc + jnp.sum((blk > x).astype(jnp.int32), axis=0, keepdims=True)
    out_ref[...] = acc


@functools.cache
def _make_tc_call():
    return pl.pallas_call(
        _tc_body,
        in_specs=[
            pl.BlockSpec(memory_space=pltpu.SMEM),
            pl.BlockSpec(memory_space=pltpu.HBM),
        ],
        out_specs=pl.BlockSpec(memory_space=pltpu.VMEM),
        out_shape=jax.ShapeDtypeStruct((1, B), jnp.int32),
        scratch_shapes=[pltpu.VMEM((B, B), jnp.float32), pltpu.SemaphoreType.DMA]
        + [pltpu.VMEM((CV, B), jnp.float32) for _ in range(NBLK)]
        + [pltpu.SemaphoreType.DMA for _ in range(NBLK)],
    )


def _sc_body(flat_ref, tok_ref, out_ref, tok_v, idx_v, x_v, buf, out_v, sem):
    c = lax.axis_index("c")
    s = lax.axis_index("s")
    wid = s * 2 + c
    row0 = VT + wid * RW
    pltpu.sync_copy(tok_ref, tok_v)
    lane = lax.iota(jnp.int32, 16)
    for g in range(G):
        t = tok_v[pl.ds(16 * g, 16)]
        idx_v[pl.ds(16 * g, 16)] = t * B + (lane + 16 * g)
    pltpu.async_copy(flat_ref.at[idx_v], x_v, sem).wait()
    xs = [x_v[pl.ds(16 * g, 16)] for g in range(G)]
    accs = [jnp.zeros((16,), jnp.int32) for _ in range(G)]
    one = jnp.ones((16,), jnp.int32)
    zero = jnp.zeros((16,), jnp.int32)
    U = 4
    for ch in range(NCH):
        elem0 = (row0 + ch * CHR) * B
        pltpu.sync_copy(flat_ref.at[pl.ds(elem0, CHR * B)], buf)

        def body(i, carry):
            out = list(carry)
            for u in range(U):
                base = (i * U + u) * B
                for g in range(G):
                    v = buf[pl.ds(base + 16 * g, 16)]
                    out[g] = out[g] + jnp.where(v > xs[g], one, zero)
            return tuple(out)

        accs = list(lax.fori_loop(0, CHR // U, body, tuple(accs)))
    for g in range(G):
        out_v[pl.ds(16 * g, 16)] = accs[g]
    pltpu.sync_copy(out_v, out_ref.at[pl.ds(wid * B, B)])


@functools.cache
def _make_sc_call():
    mesh = plsc.VectorSubcoreMesh(core_axis_name="c", subcore_axis_name="s")
    return pl.kernel(
        _sc_body,
        mesh=mesh,
        out_type=jax.ShapeDtypeStruct((NW * B,), jnp.int32),
        scratch_types=[
            pltpu.VMEM((B,), jnp.int32),
            pltpu.VMEM((B,), jnp.int32),
            pltpu.VMEM((B,), jnp.float32),
            pltpu.VMEM((CHR * B,), jnp.float32),
            pltpu.VMEM((B,), jnp.int32),
            pltpu.SemaphoreType.DMA,
        ],
    )


def kernel(logits, token_ids):
    tok = token_ids.astype(jnp.int32)
    view = logits.T
    flat = view.reshape(-1)
    tc_counts = _make_tc_call()(tok, view)
    sc_counts = _make_sc_call()(flat, tok)
    total = tc_counts.reshape(B) + sc_counts.reshape(NW, B).sum(axis=0)
    return total.astype(jnp.int64)


# speedup vs baseline: 1.0292x; 1.0292x over previous
import functools

import jax
import jax.numpy as jnp
from jax import lax
from jax.experimental import pallas as pl
from jax.experimental.pallas import tpu as pltpu
from jax.experimental.pallas import tpu_sc as plsc

B = 128
V = 100000

VSC = 12800
VT = V - VSC

NBLK = 20
CV = VT // NBLK
assert CV * NBLK == VT and CV % 8 == 0

NW = 32
RW = VSC // NW
NCH = 2
CHR = RW // NCH
assert RW * NW == VSC and CHR * NCH == RW
G = B // 16


def _tc_body(tok_ref, hbm_ref, out_ref, win_ref, wsem, *scratch):
    bufs = scratch[:NBLK]
    sems = scratch[NBLK:]
    wdescs = []
    for b in range(B):
        t = tok_ref[b]
        d = pltpu.make_async_copy(
            hbm_ref.at[pl.ds(t, 1), :], win_ref.at[pl.ds(b, 1), :], wsem
        )
        d.start()
        wdescs.append(d)
    descs = [
        pltpu.make_async_copy(hbm_ref.at[pl.ds(j * CV, CV), :], bufs[j], sems[j])
        for j in range(NBLK)
    ]
    for d in descs:
        d.start()
    for d in wdescs:
        d.wait()
    lane = lax.broadcasted_iota(jnp.int32, (1, B), 1)
    x = jnp.zeros((1, B), jnp.float32)
    for b in range(B):
        x = x + jnp.where(lane == b, win_ref[pl.ds(b, 1), :], 0.0)
    acc = jnp.zeros((1, B), jnp.int32)
    for j in range(NBLK):
        descs[j].wait()
        blk = bufs[j][...]
        acc = acc + jnp.sum((blk > x).astype(jnp.int32), axis=0, keepdims=True)
    out_ref[...] = acc


@functools.cache
def _make_tc_call():
    return pl.pallas_call(
        _tc_body,
        in_specs=[
            pl.BlockSpec(memory_space=pltpu.SMEM),
            pl.BlockSpec(memory_space=pltpu.HBM),
        ],
        out_specs=pl.BlockSpec(memory_space=pltpu.VMEM),
        out_shape=jax.ShapeDtypeStruct((1, B), jnp.int32),
        scratch_shapes=[pltpu.VMEM((B, B), jnp.float32), pltpu.SemaphoreType.DMA]
        + [pltpu.VMEM((CV, B), jnp.float32) for _ in range(NBLK)]
        + [pltpu.SemaphoreType.DMA for _ in range(NBLK)],
    )


def _sc_body(
    flat_ref, tok_ref, out_ref, tok_v, idx_v, x_v, buf0, buf1, out_v, gsem, sem0, sem1
):
    c = lax.axis_index("c")
    s = lax.axis_index("s")
    wid = s * 2 + c
    row0 = VT + wid * RW
    bufs = [buf0, buf1]
    sems = [sem0, sem1]
    d0 = pltpu.make_async_copy(
        flat_ref.at[pl.ds(row0 * B, CHR * B)], bufs[0], sems[0]
    )
    d0.start()
    pltpu.sync_copy(tok_ref, tok_v)
    lane = lax.iota(jnp.int32, 16)
    for g in range(G):
        t = tok_v[pl.ds(16 * g, 16)]
        idx_v[pl.ds(16 * g, 16)] = t * B + (lane + 16 * g)
    pltpu.async_copy(flat_ref.at[idx_v], x_v, gsem).wait()
    xs = [x_v[pl.ds(16 * g, 16)] for g in range(G)]
    accs = [jnp.zeros((16,), jnp.int32) for _ in range(G)]
    one = jnp.ones((16,), jnp.int32)
    zero = jnp.zeros((16,), jnp.int32)
    U = 4
    descs = [d0, None]
    for ch in range(NCH):
        if ch + 1 < NCH:
            nxt = (ch + 1) % 2
            descs[nxt] = pltpu.make_async_copy(
                flat_ref.at[pl.ds((row0 + (ch + 1) * CHR) * B, CHR * B)],
                bufs[nxt],
                sems[nxt],
            )
            descs[nxt].start()
        descs[ch % 2].wait()
        buf = bufs[ch % 2]

        def body(i, carry):
            out = list(carry)
            for u in range(U):
                base = (i * U + u) * B
                for g in range(G):
                    v = buf[pl.ds(base + 16 * g, 16)]
                    out[g] = out[g] + jnp.where(v > xs[g], one, zero)
            return tuple(out)

        accs = list(lax.fori_loop(0, CHR // U, body, tuple(accs)))
    for g in range(G):
        out_v[pl.ds(16 * g, 16)] = accs[g]
    pltpu.sync_copy(out_v, out_ref.at[pl.ds(wid * B, B)])


@functools.cache
def _make_sc_call():
    mesh = plsc.VectorSubcoreMesh(core_axis_name="c", subcore_axis_name="s")
    return pl.kernel(
        _sc_body,
        mesh=mesh,
        out_type=jax.ShapeDtypeStruct((NW * B,), jnp.int32),
        scratch_types=[
            pltpu.VMEM((B,), jnp.int32),
            pltpu.VMEM((B,), jnp.int32),
            pltpu.VMEM((B,), jnp.float32),
            pltpu.VMEM((CHR * B,), jnp.float32),
            pltpu.VMEM((CHR * B,), jnp.float32),
            pltpu.VMEM((B,), jnp.int32),
            pltpu.SemaphoreType.DMA,
            pltpu.SemaphoreType.DMA,
            pltpu.SemaphoreType.DMA,
        ],
    )


def kernel(logits, token_ids):
    tok = token_ids.astype(jnp.int32)
    view = logits.T
    flat = view.reshape(-1)
    sc_counts = _make_sc_call()(flat, tok)
    tc_counts = _make_tc_call()(tok, view)
    total = tc_counts.reshape(B) + sc_counts.reshape(NW, B).sum(axis=0)
    return total.astype(jnp.int64)


# speedup vs baseline: 1.9097x; 1.8555x over previous
import functools

import jax
import jax.numpy as jnp
from jax import lax
from jax.experimental import pallas as pl
from jax.experimental.pallas import tpu as pltpu

B = 128
V = 100000
NBLK = 20
CV = V // NBLK
assert CV * NBLK == V and CV % 8 == 0


def _count_body(tok_ref, hbm_ref, out_ref, win_ref, wsem, *scratch):
    bufs = scratch[:NBLK]
    sems = scratch[NBLK:]
    wdescs = []
    for b in range(B):
        t = tok_ref[b]
        d = pltpu.make_async_copy(
            hbm_ref.at[pl.ds(t, 1), :], win_ref.at[pl.ds(b, 1), :], wsem
        )
        d.start()
        wdescs.append(d)
    descs = [
        pltpu.make_async_copy(hbm_ref.at[pl.ds(j * CV, CV), :], bufs[j], sems[j])
        for j in range(NBLK)
    ]
    for d in descs:
        d.start()
    for d in wdescs:
        d.wait()
    lane = lax.broadcasted_iota(jnp.int32, (1, B), 1)
    x = jnp.zeros((1, B), jnp.float32)
    for b in range(B):
        x = x + jnp.where(lane == b, win_ref[pl.ds(b, 1), :], 0.0)
    acc = jnp.zeros((1, B), jnp.int32)
    for j in range(NBLK):
        descs[j].wait()
        blk = bufs[j][...]
        acc = acc + jnp.sum((blk > x).astype(jnp.int32), axis=0, keepdims=True)
    out_ref[...] = acc


@functools.cache
def _make_count_call():
    return pl.pallas_call(
        _count_body,
        in_specs=[
            pl.BlockSpec(memory_space=pltpu.SMEM),
            pl.BlockSpec(memory_space=pltpu.HBM),
        ],
        out_specs=pl.BlockSpec(memory_space=pltpu.VMEM),
        out_shape=jax.ShapeDtypeStruct((1, B), jnp.int32),
        scratch_shapes=[pltpu.VMEM((B, B), jnp.float32), pltpu.SemaphoreType.DMA]
        + [pltpu.VMEM((CV, B), jnp.float32) for _ in range(NBLK)]
        + [pltpu.SemaphoreType.DMA for _ in range(NBLK)],
    )


def kernel(logits, token_ids):
    tok = token_ids.astype(jnp.int32)
    counts = _make_count_call()(tok, logits.T)
    return counts.reshape(B).astype(jnp.int64)


# speedup vs baseline: 1.9535x; 1.0229x over previous
import functools

import jax
import jax.numpy as jnp
from jax import lax
from jax.experimental import pallas as pl
from jax.experimental.pallas import tpu as pltpu

B = 128
V = 100000
NBLK = 50
CV = V // NBLK
assert CV * NBLK == V and CV % 8 == 0


def _count_body(tok_ref, hbm_ref, out_ref, win_ref, wsem, *scratch):
    bufs = scratch[:NBLK]
    sems = scratch[NBLK:]
    wdescs = []
    for b in range(B):
        t = tok_ref[b]
        d = pltpu.make_async_copy(
            hbm_ref.at[pl.ds(t, 1), :], win_ref.at[pl.ds(b, 1), :], wsem
        )
        d.start()
        wdescs.append(d)
    descs = [
        pltpu.make_async_copy(hbm_ref.at[pl.ds(j * CV, CV), :], bufs[j], sems[j])
        for j in range(NBLK)
    ]
    for d in descs:
        d.start()
    for d in wdescs:
        d.wait()
    lane = lax.broadcasted_iota(jnp.int32, (1, B), 1)
    x = jnp.zeros((1, B), jnp.float32)
    for b in range(B):
        x = x + jnp.where(lane == b, win_ref[pl.ds(b, 1), :], 0.0)
    acc = jnp.zeros((1, B), jnp.int32)
    for j in range(NBLK):
        descs[j].wait()
        blk = bufs[j][...]
        acc = acc + jnp.sum((blk > x).astype(jnp.int32), axis=0, keepdims=True)
    out_ref[...] = acc


@functools.cache
def _make_count_call():
    return pl.pallas_call(
        _count_body,
        in_specs=[
            pl.BlockSpec(memory_space=pltpu.SMEM),
            pl.BlockSpec(memory_space=pltpu.HBM),
        ],
        out_specs=pl.BlockSpec(memory_space=pltpu.VMEM),
        out_shape=jax.ShapeDtypeStruct((1, B), jnp.int32),
        scratch_shapes=[pltpu.VMEM((B, B), jnp.float32), pltpu.SemaphoreType.DMA]
        + [pltpu.VMEM((CV, B), jnp.float32) for _ in range(NBLK)]
        + [pltpu.SemaphoreType.DMA for _ in range(NBLK)],
    )


def kernel(logits, token_ids):
    tok = token_ids.astype(jnp.int32)
    counts = _make_count_call()(tok, logits.T)
    return counts.reshape(B).astype(jnp.int64)
